# baseline (speedup 1.0000x reference)
import functools

import jax
import jax.numpy as jnp
from jax import lax
from jax.experimental import pallas as pl
from jax.experimental.pallas import tpu as pltpu
from jax.experimental.pallas import tpu_sc as plsc

NC = 2
NS = 16
NW = NC * NS
L = 16

BATCH = 16384
D = 64
B_PER_W = BATCH // NW
CHUNK = 128
NCHUNK = B_PER_W // CHUNK


def _fcf_body(user_hbm, item_hbm, utab_hbm, itab_hbm, params_hbm, out_hbm,
              uidx_v, iidx_v, urows_v, irows_v, params_v, out_v, mat_v,
              usem, isem):
    wid = lax.axis_index("s") * NC + lax.axis_index("c")
    base = wid * B_PER_W

    pltpu.sync_copy(user_hbm.at[wid], uidx_v)
    pltpu.sync_copy(item_hbm.at[wid], iidx_v)
    pltpu.sync_copy(params_hbm, params_v)

    ucopies = []
    icopies = []
    for j in range(NCHUNK):
        ucopies.append(pltpu.async_copy(
            utab_hbm.at[uidx_v.at[j]], urows_v.at[pl.ds(j * CHUNK, CHUNK)],
            usem))
        icopies.append(pltpu.async_copy(
            itab_hbm.at[iidx_v.at[j]], irows_v.at[pl.ds(j * CHUNK, CHUNK)],
            isem))
    for c in ucopies + icopies:
        c.wait()

    w0 = params_v[pl.ds(0, L)]
    w1 = params_v[pl.ds(L, L)]
    w2 = params_v[pl.ds(2 * L, L)]
    w3 = params_v[pl.ds(3 * L, L)]
    bias_splat = jnp.full((L,), params_v[pl.ds(D, L)][0], jnp.float32)
    iota = lax.iota(jnp.int32, L)

    def body(g, carry):
        b0 = g * L
        for bb in range(L):
            b = b0 + bb
            acc = urows_v[b, pl.ds(0, L)] * irows_v[b, pl.ds(0, L)] * w0
            acc = acc + urows_v[b, pl.ds(L, L)] * irows_v[b, pl.ds(L, L)] * w1
            acc = acc + urows_v[b, pl.ds(2 * L, L)] * irows_v[b, pl.ds(2 * L, L)] * w2
            acc = acc + urows_v[b, pl.ds(3 * L, L)] * irows_v[b, pl.ds(3 * L, L)] * w3
            mat_v[bb, :] = acc
        colsum = bias_splat
        for c in range(L):
            colsum = colsum + plsc.load_gather(
                mat_v, [iota, jnp.full((L,), c, jnp.int32)])
        out_v[pl.ds(b0, L)] = colsum
        return carry

    lax.fori_loop(0, B_PER_W // L, body, 0)

    pltpu.sync_copy(out_v, out_hbm.at[pl.ds(base, B_PER_W)])


def kernel(user, item, users_embeddings, items_embeddings, affine_w, affine_b):
    user_r = user.astype(jnp.int32).reshape(NW, NCHUNK, CHUNK)
    item_r = item.astype(jnp.int32).reshape(NW, NCHUNK, CHUNK)
    params = jnp.concatenate(
        [affine_w.reshape(-1), affine_b.reshape(-1),
         jnp.zeros((15,), jnp.float32)])

    mesh = plsc.VectorSubcoreMesh(core_axis_name="c", subcore_axis_name="s")
    fcf = functools.partial(
        pl.kernel,
        mesh=mesh,
        compiler_params=pltpu.CompilerParams(
            needs_layout_passes=False, use_tc_tiling_on_sc=False),
        out_type=jax.ShapeDtypeStruct((BATCH,), jnp.float32),
        scratch_types=[
            pltpu.VMEM((NCHUNK, CHUNK), jnp.int32),
            pltpu.VMEM((NCHUNK, CHUNK), jnp.int32),
            pltpu.VMEM((B_PER_W, D), jnp.float32),
            pltpu.VMEM((B_PER_W, D), jnp.float32),
            pltpu.VMEM((80,), jnp.float32),
            pltpu.VMEM((B_PER_W,), jnp.float32),
            pltpu.VMEM((L, L), jnp.float32),
            pltpu.SemaphoreType.DMA,
            pltpu.SemaphoreType.DMA,
        ],
    )(_fcf_body)
    return fcf(user_r, item_r, users_embeddings, items_embeddings, params)
